# baseline (speedup 1.0000x reference)
import functools

import numpy as np
import jax
import jax.numpy as jnp
from jax import lax
from jax.experimental import pallas as pl
from jax.experimental.pallas import tpu as pltpu
from jax.experimental.pallas import tpu_sc as plsc

_NSUB = 512
_K = 32
_SUB = 256
_T = 128
_NEG = -1.0e30
_BIG = 1 << 20
_SPLIT = 2432
_NC, _NS, _L = 2, 16, 16
_NW = _NC * _NS

_rows, _cols = [], []
for _i in range(_K):
    for _j in range(_K // (_i + 1)):
        _rows.append(_i)
        _cols.append(_j)
_NREAL = len(_rows)
_NCAND = 128
_CNT = [_K // (_i + 1) for _i in range(_K)]
_ROWOF = np.full((_NCAND, 1), _K - 1, np.int32)
for _s in range(_NREAL):
    _ROWOF[_s, 0] = _rows[_s]


def _topk32(x):
    iota = lax.broadcasted_iota(jnp.int32, x.shape, 0)
    vals, idxs = [], []
    for _ in range(_K):
        m = jnp.max(x, axis=0, keepdims=True)
        mask = x == m
        smin = jnp.min(jnp.where(mask, iota, _BIG), axis=0, keepdims=True)
        vals.append(m)
        idxs.append(smin)
        x = jnp.where(iota == smin, _NEG, x)
    return jnp.concatenate(vals, 0), jnp.concatenate(idxs, 0)


def _body(q_ref, kl_ref, kr_ref, ro_ref, w_ref, idx_ref):
    q = q_ref[0]
    kl = kl_ref[0]
    kr = kr_ref[0]
    dn = (((1,), (1,)), ((), ()))
    sl = lax.dot_general(kl, q[:, :_SUB], dn, preferred_element_type=jnp.float32)
    sr = lax.dot_general(kr, q[:, _SUB:], dn, preferred_element_type=jnp.float32)

    lv, li = _topk32(sl)
    rv, ri = _topk32(sr)

    pieces = []
    for i in range(_K):
        c = _CNT[i]
        if c > 1:
            pieces.append(lv[i:i + 1, :] + rv[0:c, :])
        else:
            pieces.append(lv[i:, :] + rv[0:1, :])
            break
    pieces.append(jnp.full((_NCAND - _NREAL, lv.shape[1]), _NEG, jnp.float32))
    cand = jnp.concatenate(pieces, axis=0)

    iota2 = lax.broadcasted_iota(jnp.int32, cand.shape, 0)
    rowmat = ro_ref[...] + jnp.zeros(cand.shape, jnp.int32)
    vals2, rows2 = [], []
    for _ in range(_K):
        m = jnp.max(cand, axis=0, keepdims=True)
        mask = cand == m
        smin = jnp.min(jnp.where(mask, iota2, _BIG), axis=0, keepdims=True)
        rmin = jnp.min(jnp.where(mask, rowmat, _BIG), axis=0, keepdims=True)
        vals2.append(m)
        rows2.append(rmin)
        cand = jnp.where(iota2 == smin, _NEG, cand)
    v2 = jnp.concatenate(vals2, 0)
    r2 = jnp.concatenate(rows2, 0)

    e = jnp.exp(v2 - v2[0:1, :])
    w = e / jnp.sum(e, axis=0, keepdims=True)

    comb = li * _NSUB + ri
    acc = jnp.zeros(r2.shape, jnp.int32)
    for i in range(_K):
        acc = acc + jnp.where(r2 == jnp.int32(i), comb[i:i + 1, :], 0)

    w_ref[0] = w
    idx_ref[0] = acc


def _tc_retrieve(q_part, keyl, keyr):
    n, toks, d = q_part.shape
    grid = (n, toks // _T)
    return pl.pallas_call(
        _body,
        grid=grid,
        in_specs=[
            pl.BlockSpec((1, _T, d), lambda h, t: (h, t, 0)),
            pl.BlockSpec((1, _NSUB, _SUB), lambda h, t: (h, 0, 0)),
            pl.BlockSpec((1, _NSUB, _SUB), lambda h, t: (h, 0, 0)),
            pl.BlockSpec((_NCAND, 1), lambda h, t: (0, 0)),
        ],
        out_specs=[
            pl.BlockSpec((1, _K, _T), lambda h, t: (h, 0, t)),
            pl.BlockSpec((1, _K, _T), lambda h, t: (h, 0, t)),
        ],
        out_shape=[
            jax.ShapeDtypeStruct((n, _K, toks), jnp.float32),
            jax.ShapeDtypeStruct((n, _K, toks), jnp.int32),
        ],
    )(q_part, keyl, keyr, jnp.asarray(_ROWOF))


def _score_body(q_ref, kl_ref, kr_ref, sl_ref, sr_ref):
    q = q_ref[0]
    dn = (((1,), (1,)), ((), ()))
    sl_ref[0] = lax.dot_general(q[:, :_SUB], kl_ref[0], dn,
                                preferred_element_type=jnp.float32)
    sr_ref[0] = lax.dot_general(q[:, _SUB:], kr_ref[0], dn,
                                preferred_element_type=jnp.float32)


def _tc_scores(q_part, keyl, keyr):
    n, toks, d = q_part.shape
    grid = (n, toks // _T)
    return pl.pallas_call(
        _score_body,
        grid=grid,
        in_specs=[
            pl.BlockSpec((1, _T, d), lambda h, t: (h, t, 0)),
            pl.BlockSpec((1, _NSUB, _SUB), lambda h, t: (h, 0, 0)),
            pl.BlockSpec((1, _NSUB, _SUB), lambda h, t: (h, 0, 0)),
        ],
        out_specs=[
            pl.BlockSpec((1, _T, _NSUB), lambda h, t: (h, t, 0)),
            pl.BlockSpec((1, _T, _NSUB), lambda h, t: (h, t, 0)),
        ],
        out_shape=[
            jax.ShapeDtypeStruct((n, toks, _NSUB), jnp.float32),
            jax.ShapeDtypeStruct((n, toks, _NSUB), jnp.float32),
        ],
    )(q_part, keyl, keyr)


def _sc_retrieve(sl, sr):
    n, S, _ = sl.shape
    nbh = S // _L
    nb_total = n * nbh
    nb_w = nb_total // _NW
    rowof = np.zeros((_NCAND,), np.int32)
    for s in range(_NREAL):
        rowof[s] = _rows[s]
    rowof[_NREAL:] = _K - 1

    @functools.partial(
        pl.kernel,
        out_type=[jax.ShapeDtypeStruct((n, S, _K), jnp.float32),
                  jax.ShapeDtypeStruct((n, S, _K), jnp.int32)],
        mesh=plsc.VectorSubcoreMesh(core_axis_name="c", subcore_axis_name="s"),
        compiler_params=pltpu.CompilerParams(needs_layout_passes=False,
                                             use_tc_tiling_on_sc=False),
        scratch_types=[
            pltpu.VMEM((_L, _NSUB), jnp.float32),
            pltpu.VMEM((32 * _L,), jnp.float32),
            pltpu.VMEM((_K * _L,), jnp.float32),
            pltpu.VMEM((_K * _L,), jnp.int32),
            pltpu.VMEM((_K * _L,), jnp.float32),
            pltpu.VMEM((_K * _L,), jnp.int32),
            pltpu.VMEM((_NCAND * _L,), jnp.float32),
            pltpu.VMEM((8 * _L,), jnp.float32),
            pltpu.VMEM((_K * _L,), jnp.int32),
            pltpu.VMEM((_K * _L,), jnp.float32),
            pltpu.VMEM((_L, _K), jnp.float32),
            pltpu.VMEM((_L, _K), jnp.int32),
            pltpu.VMEM((_NCAND,), jnp.int32),
        ],
    )
    def sc_k(sl_hbm, sr_hbm, ro_hbm, w_hbm, i_hbm,
             x, M, LV, LI, RV, RI, CD, M2, CB, W, WT, IT, RO):
        lane = lax.iota(jnp.int32, _L)
        negv = jnp.full((_L,), _NEG, jnp.float32)
        pltpu.sync_copy(ro_hbm, RO)
        wid = lax.axis_index("s") * _NC + lax.axis_index("c")

        def splat(v):
            return jnp.full((_L,), v, jnp.int32)

        def argmax_tree(pairs):
            while len(pairs) > 1:
                nxt = []
                for a in range(0, len(pairs) - 1, 2):
                    (v1, i1), (v2, i2) = pairs[a], pairs[a + 1]
                    gt = v2 > v1
                    nxt.append((jnp.where(gt, v2, v1), jnp.where(gt, i2, i1)))
                if len(pairs) % 2:
                    nxt.append(pairs[-1])
                pairs = nxt
            return pairs[0]

        def topk_side(OV, OI):
            def cinit(ci, _):
                m = plsc.load_gather(x, [lane, splat(ci * _L)])
                for j in range(1, _L):
                    m = jnp.maximum(m, plsc.load_gather(x, [lane, splat(ci * _L + j)]))
                plsc.store_scatter(M, [splat(ci) * _L + lane], m)
                return 0
            lax.fori_loop(0, _NSUB // _L, cinit, 0, unroll=False)

            def step(kk, _):
                mv, csel = argmax_tree(
                    [(M[pl.ds(ci * _L, _L)], splat(ci)) for ci in range(32)])
                base = csel * _L
                vals = [plsc.load_gather(x, [lane, base + j]) for j in range(_L)]
                _, rsel = argmax_tree(
                    [(vals[j], splat(j)) for j in range(_L)])
                plsc.store_scatter(x, [lane, base + rsel], negv)
                nm = negv
                for j in range(_L):
                    nm = jnp.maximum(nm, jnp.where(rsel == j, negv, vals[j]))
                plsc.store_scatter(M, [csel * _L + lane], nm)
                plsc.store_scatter(OV, [splat(kk) * _L + lane], mv)
                plsc.store_scatter(OI, [splat(kk) * _L + lane], base + rsel)
                return 0
            lax.fori_loop(0, _K, step, 0, unroll=False)

        def batch(bk, _):
            bi = wid * nb_w + bk
            h = bi // nbh
            t0 = (bi % nbh) * _L
            pltpu.sync_copy(sl_hbm.at[h, pl.ds(t0, _L), :], x)
            topk_side(LV, LI)
            pltpu.sync_copy(sr_hbm.at[h, pl.ds(t0, _L), :], x)
            topk_side(RV, RI)

            for s in range(_NREAL):
                CD[pl.ds(s * _L, _L)] = (LV[pl.ds(_rows[s] * _L, _L)]
                                         + RV[pl.ds(_cols[s] * _L, _L)])
            for s in range(_NREAL, _NCAND):
                CD[pl.ds(s * _L, _L)] = negv
            for ci in range(_NCAND // _L):
                m = CD[pl.ds(ci * _L * _L, _L)]
                for j in range(1, _L):
                    m = jnp.maximum(m, CD[pl.ds((ci * _L + j) * _L, _L)])
                M2[pl.ds(ci * _L, _L)] = m
            for i in range(_K):
                CB[pl.ds(i * _L, _L)] = LI[pl.ds(i * _L, _L)] * _NSUB + RI[pl.ds(i * _L, _L)]

            def step2(kk, _):
                mv, csel = argmax_tree(
                    [(M2[pl.ds(ci * _L, _L)], splat(ci))
                     for ci in range(_NCAND // _L)])
                base = csel * (_L * _L) + lane
                vals = [plsc.load_gather(CD, [base + j * _L]) for j in range(_L)]
                _, rsel = argmax_tree(
                    [(vals[j], splat(j)) for j in range(_L)])
                plsc.store_scatter(CD, [base + rsel * _L], negv)
                nm = negv
                for j in range(_L):
                    nm = jnp.maximum(nm, jnp.where(rsel == j, negv, vals[j]))
                plsc.store_scatter(M2, [csel * _L + lane], nm)
                slot = csel * _L + rsel
                row = plsc.load_gather(RO, [slot])
                comb = plsc.load_gather(CB, [row * _L + lane])
                plsc.store_scatter(W, [splat(kk) * _L + lane], mv)
                plsc.store_scatter(IT, [lane, splat(kk)], comb)
                return 0
            lax.fori_loop(0, _K, step2, 0, unroll=False)

            m0 = W[pl.ds(0, _L)]
            tot = jnp.zeros((_L,), jnp.float32)
            es = []
            for i in range(_K):
                e = jnp.exp(W[pl.ds(i * _L, _L)] - m0)
                es.append(e)
                tot = tot + e
            inv = 1.0 / tot
            for i in range(_K):
                plsc.store_scatter(WT, [lane, splat(i)], es[i] * inv)

            pltpu.sync_copy(WT, w_hbm.at[h, pl.ds(t0, _L), :])
            pltpu.sync_copy(IT, i_hbm.at[h, pl.ds(t0, _L), :])
            return 0

        lax.fori_loop(0, nb_w, batch, 0, unroll=False)

    return sc_k(sl, sr, jnp.asarray(rowof))


def kernel(query, keyl, keyr):
    b, c, n, d = query.shape
    tokens = b * c
    q2 = query.reshape(tokens, n, d).transpose(1, 0, 2)
    s = _SPLIT
    t_tc = tokens - s
    parts_w, parts_i = [], []
    if t_tc:
        w1, i1 = _tc_retrieve(q2[:, :t_tc], keyl, keyr)
        parts_w.append(w1.transpose(2, 0, 1))
        parts_i.append(i1.transpose(2, 0, 1))
    if s:
        sl, sr = _tc_scores(q2[:, t_tc:], keyl, keyr)
        w2, i2 = _sc_retrieve(sl, sr)
        parts_w.append(w2.transpose(1, 0, 2))
        parts_i.append(i2.transpose(1, 0, 2))
    w = parts_w[0] if len(parts_w) == 1 else jnp.concatenate(parts_w, axis=0)
    idx = parts_i[0] if len(parts_i) == 1 else jnp.concatenate(parts_i, axis=0)
    w = w.reshape(b, c, n, _K)
    idx = idx.reshape(b, c, n, _K)
    return w, idx


# speedup vs baseline: 1.0312x; 1.0312x over previous
import functools

import numpy as np
import jax
import jax.numpy as jnp
from jax import lax
from jax.experimental import pallas as pl
from jax.experimental.pallas import tpu as pltpu
from jax.experimental.pallas import tpu_sc as plsc

_NSUB = 512
_K = 32
_SUB = 256
_T = 128
_NEG = -1.0e30
_BIG = 1 << 20
_SPLIT = 2304
_NC, _NS, _L = 2, 16, 16
_NW = _NC * _NS

_rows, _cols = [], []
for _i in range(_K):
    for _j in range(_K // (_i + 1)):
        _rows.append(_i)
        _cols.append(_j)
_NREAL = len(_rows)
_NCAND = 128
_CNT = [_K // (_i + 1) for _i in range(_K)]
_ROWOF = np.full((_NCAND, 1), _K - 1, np.int32)
for _s in range(_NREAL):
    _ROWOF[_s, 0] = _rows[_s]


def _topk32(x):
    iota = lax.broadcasted_iota(jnp.int32, x.shape, 0)
    vals, idxs = [], []
    for _ in range(_K):
        m = jnp.max(x, axis=0, keepdims=True)
        mask = x == m
        smin = jnp.min(jnp.where(mask, iota, _BIG), axis=0, keepdims=True)
        vals.append(m)
        idxs.append(smin)
        x = jnp.where(iota == smin, _NEG, x)
    return jnp.concatenate(vals, 0), jnp.concatenate(idxs, 0)


def _body(q_ref, kl_ref, kr_ref, ro_ref, w_ref, idx_ref):
    q = q_ref[0]
    kl = kl_ref[0]
    kr = kr_ref[0]
    dn = (((1,), (1,)), ((), ()))
    sl = lax.dot_general(kl, q[:, :_SUB], dn, preferred_element_type=jnp.float32)
    sr = lax.dot_general(kr, q[:, _SUB:], dn, preferred_element_type=jnp.float32)

    lv, li = _topk32(sl)
    rv, ri = _topk32(sr)

    pieces = []
    for i in range(_K):
        c = _CNT[i]
        if c > 1:
            pieces.append(lv[i:i + 1, :] + rv[0:c, :])
        else:
            pieces.append(lv[i:, :] + rv[0:1, :])
            break
    pieces.append(jnp.full((_NCAND - _NREAL, lv.shape[1]), _NEG, jnp.float32))
    cand = jnp.concatenate(pieces, axis=0)

    iota2 = lax.broadcasted_iota(jnp.int32, cand.shape, 0)
    rowmat = ro_ref[...] + jnp.zeros(cand.shape, jnp.int32)
    vals2, rows2 = [], []
    for _ in range(_K):
        m = jnp.max(cand, axis=0, keepdims=True)
        mask = cand == m
        smin = jnp.min(jnp.where(mask, iota2, _BIG), axis=0, keepdims=True)
        rmin = jnp.min(jnp.where(mask, rowmat, _BIG), axis=0, keepdims=True)
        vals2.append(m)
        rows2.append(rmin)
        cand = jnp.where(iota2 == smin, _NEG, cand)
    v2 = jnp.concatenate(vals2, 0)
    r2 = jnp.concatenate(rows2, 0)

    e = jnp.exp(v2 - v2[0:1, :])
    w = e / jnp.sum(e, axis=0, keepdims=True)

    comb = li * _NSUB + ri
    acc = jnp.zeros(r2.shape, jnp.int32)
    for i in range(_K):
        acc = acc + jnp.where(r2 == jnp.int32(i), comb[i:i + 1, :], 0)

    w_ref[0] = w
    idx_ref[0] = acc


def _tc_retrieve(q_part, keyl, keyr):
    n, toks, d = q_part.shape
    grid = (n, toks // _T)
    return pl.pallas_call(
        _body,
        grid=grid,
        in_specs=[
            pl.BlockSpec((1, _T, d), lambda h, t: (h, t, 0)),
            pl.BlockSpec((1, _NSUB, _SUB), lambda h, t: (h, 0, 0)),
            pl.BlockSpec((1, _NSUB, _SUB), lambda h, t: (h, 0, 0)),
            pl.BlockSpec((_NCAND, 1), lambda h, t: (0, 0)),
        ],
        out_specs=[
            pl.BlockSpec((1, _K, _T), lambda h, t: (h, 0, t)),
            pl.BlockSpec((1, _K, _T), lambda h, t: (h, 0, t)),
        ],
        out_shape=[
            jax.ShapeDtypeStruct((n, _K, toks), jnp.float32),
            jax.ShapeDtypeStruct((n, _K, toks), jnp.int32),
        ],
    )(q_part, keyl, keyr, jnp.asarray(_ROWOF))


def _score_body(q_ref, kl_ref, kr_ref, sl_ref, sr_ref):
    q = q_ref[0]
    dn = (((1,), (1,)), ((), ()))
    sl_ref[0] = lax.dot_general(q[:, :_SUB], kl_ref[0], dn,
                                preferred_element_type=jnp.float32)
    sr_ref[0] = lax.dot_general(q[:, _SUB:], kr_ref[0], dn,
                                preferred_element_type=jnp.float32)


def _tc_scores(q_part, keyl, keyr):
    n, toks, d = q_part.shape
    grid = (n, toks // _T)
    return pl.pallas_call(
        _score_body,
        grid=grid,
        in_specs=[
            pl.BlockSpec((1, _T, d), lambda h, t: (h, t, 0)),
            pl.BlockSpec((1, _NSUB, _SUB), lambda h, t: (h, 0, 0)),
            pl.BlockSpec((1, _NSUB, _SUB), lambda h, t: (h, 0, 0)),
        ],
        out_specs=[
            pl.BlockSpec((1, _T, _NSUB), lambda h, t: (h, t, 0)),
            pl.BlockSpec((1, _T, _NSUB), lambda h, t: (h, t, 0)),
        ],
        out_shape=[
            jax.ShapeDtypeStruct((n, toks, _NSUB), jnp.float32),
            jax.ShapeDtypeStruct((n, toks, _NSUB), jnp.float32),
        ],
    )(q_part, keyl, keyr)


def _sc_retrieve(sl, sr):
    n, S, _ = sl.shape
    nbh = S // _L
    nb_total = n * nbh
    nb_w = nb_total // _NW
    rowof = np.zeros((_NCAND,), np.int32)
    for s in range(_NREAL):
        rowof[s] = _rows[s]
    rowof[_NREAL:] = _K - 1

    @functools.partial(
        pl.kernel,
        out_type=[jax.ShapeDtypeStruct((n, S, _K), jnp.float32),
                  jax.ShapeDtypeStruct((n, S, _K), jnp.int32)],
        mesh=plsc.VectorSubcoreMesh(core_axis_name="c", subcore_axis_name="s"),
        compiler_params=pltpu.CompilerParams(needs_layout_passes=False,
                                             use_tc_tiling_on_sc=False),
        scratch_types=[
            pltpu.VMEM((_L, _NSUB), jnp.float32),
            pltpu.VMEM((32 * _L,), jnp.float32),
            pltpu.VMEM((_K * _L,), jnp.float32),
            pltpu.VMEM((_K * _L,), jnp.int32),
            pltpu.VMEM((_K * _L,), jnp.float32),
            pltpu.VMEM((_K * _L,), jnp.int32),
            pltpu.VMEM((_NCAND * _L,), jnp.float32),
            pltpu.VMEM((8 * _L,), jnp.float32),
            pltpu.VMEM((_K * _L,), jnp.int32),
            pltpu.VMEM((_K * _L,), jnp.float32),
            pltpu.VMEM((_L, _K), jnp.float32),
            pltpu.VMEM((_L, _K), jnp.int32),
            pltpu.VMEM((_NCAND,), jnp.int32),
        ],
    )
    def sc_k(sl_hbm, sr_hbm, ro_hbm, w_hbm, i_hbm,
             x, M, LV, LI, RV, RI, CD, M2, CB, W, WT, IT, RO):
        lane = lax.iota(jnp.int32, _L)
        negv = jnp.full((_L,), _NEG, jnp.float32)
        pltpu.sync_copy(ro_hbm, RO)
        wid = lax.axis_index("s") * _NC + lax.axis_index("c")

        def splat(v):
            return jnp.full((_L,), v, jnp.int32)

        def argmax_tree(pairs):
            while len(pairs) > 1:
                nxt = []
                for a in range(0, len(pairs) - 1, 2):
                    (v1, i1), (v2, i2) = pairs[a], pairs[a + 1]
                    gt = v2 > v1
                    nxt.append((jnp.where(gt, v2, v1), jnp.where(gt, i2, i1)))
                if len(pairs) % 2:
                    nxt.append(pairs[-1])
                pairs = nxt
            return pairs[0]

        def topk_side(OV, OI):
            def cinit(ci, _):
                m = plsc.load_gather(x, [lane, splat(ci * _L)])
                for j in range(1, _L):
                    m = jnp.maximum(m, plsc.load_gather(x, [lane, splat(ci * _L + j)]))
                plsc.store_scatter(M, [splat(ci) * _L + lane], m)
                return 0
            lax.fori_loop(0, _NSUB // _L, cinit, 0, unroll=False)

            def step(kk, _):
                mv, csel = argmax_tree(
                    [(M[pl.ds(ci * _L, _L)], splat(ci)) for ci in range(32)])
                base = csel * _L
                vals = [plsc.load_gather(x, [lane, base + j]) for j in range(_L)]
                _, rsel = argmax_tree(
                    [(vals[j], splat(j)) for j in range(_L)])
                plsc.store_scatter(x, [lane, base + rsel], negv)
                nm = negv
                for j in range(_L):
                    nm = jnp.maximum(nm, jnp.where(rsel == j, negv, vals[j]))
                plsc.store_scatter(M, [csel * _L + lane], nm)
                plsc.store_scatter(OV, [splat(kk) * _L + lane], mv)
                plsc.store_scatter(OI, [splat(kk) * _L + lane], base + rsel)
                return 0
            lax.fori_loop(0, _K, step, 0, unroll=False)

        def batch(bk, _):
            bi = wid * nb_w + bk
            h = bi // nbh
            t0 = (bi % nbh) * _L
            pltpu.sync_copy(sl_hbm.at[h, pl.ds(t0, _L), :], x)
            topk_side(LV, LI)
            pltpu.sync_copy(sr_hbm.at[h, pl.ds(t0, _L), :], x)
            topk_side(RV, RI)

            for s in range(_NREAL):
                CD[pl.ds(s * _L, _L)] = (LV[pl.ds(_rows[s] * _L, _L)]
                                         + RV[pl.ds(_cols[s] * _L, _L)])
            for s in range(_NREAL, _NCAND):
                CD[pl.ds(s * _L, _L)] = negv
            for ci in range(_NCAND // _L):
                m = CD[pl.ds(ci * _L * _L, _L)]
                for j in range(1, _L):
                    m = jnp.maximum(m, CD[pl.ds((ci * _L + j) * _L, _L)])
                M2[pl.ds(ci * _L, _L)] = m
            for i in range(_K):
                CB[pl.ds(i * _L, _L)] = LI[pl.ds(i * _L, _L)] * _NSUB + RI[pl.ds(i * _L, _L)]

            def step2(kk, _):
                mv, csel = argmax_tree(
                    [(M2[pl.ds(ci * _L, _L)], splat(ci))
                     for ci in range(_NCAND // _L)])
                base = csel * (_L * _L) + lane
                vals = [plsc.load_gather(CD, [base + j * _L]) for j in range(_L)]
                _, rsel = argmax_tree(
                    [(vals[j], splat(j)) for j in range(_L)])
                plsc.store_scatter(CD, [base + rsel * _L], negv)
                nm = negv
                for j in range(_L):
                    nm = jnp.maximum(nm, jnp.where(rsel == j, negv, vals[j]))
                plsc.store_scatter(M2, [csel * _L + lane], nm)
                slot = csel * _L + rsel
                row = plsc.load_gather(RO, [slot])
                comb = plsc.load_gather(CB, [row * _L + lane])
                plsc.store_scatter(W, [splat(kk) * _L + lane], mv)
                plsc.store_scatter(IT, [lane, splat(kk)], comb)
                return 0
            lax.fori_loop(0, _K, step2, 0, unroll=False)

            m0 = W[pl.ds(0, _L)]
            tot = jnp.zeros((_L,), jnp.float32)
            es = []
            for i in range(_K):
                e = jnp.exp(W[pl.ds(i * _L, _L)] - m0)
                es.append(e)
                tot = tot + e
            inv = 1.0 / tot
            for i in range(_K):
                plsc.store_scatter(WT, [lane, splat(i)], es[i] * inv)

            pltpu.sync_copy(WT, w_hbm.at[h, pl.ds(t0, _L), :])
            pltpu.sync_copy(IT, i_hbm.at[h, pl.ds(t0, _L), :])
            return 0

        lax.fori_loop(0, nb_w, batch, 0, unroll=False)

    return sc_k(sl, sr, jnp.asarray(rowof))


def kernel(query, keyl, keyr):
    b, c, n, d = query.shape
    tokens = b * c
    q2 = query.reshape(tokens, n, d).transpose(1, 0, 2)
    s = _SPLIT
    t_tc = tokens - s
    parts_w, parts_i = [], []
    if t_tc:
        w1, i1 = _tc_retrieve(q2[:, :t_tc], keyl, keyr)
        parts_w.append(w1.transpose(2, 0, 1))
        parts_i.append(i1.transpose(2, 0, 1))
    if s:
        sl, sr = _tc_scores(q2[:, t_tc:], keyl, keyr)
        w2, i2 = _sc_retrieve(sl, sr)
        parts_w.append(w2.transpose(1, 0, 2))
        parts_i.append(i2.transpose(1, 0, 2))
    w = parts_w[0] if len(parts_w) == 1 else jnp.concatenate(parts_w, axis=0)
    idx = parts_i[0] if len(parts_i) == 1 else jnp.concatenate(parts_i, axis=0)
    w = w.reshape(b, c, n, _K)
    idx = idx.reshape(b, c, n, _K)
    return w, idx
